# baseline (device time: 117449 ns/iter reference)
import jax
import jax.numpy as jnp
from jax import lax
from jax.experimental import pallas as pl
from jax.experimental.pallas import tpu as pltpu

T = 1024
D = 1024
F = 2048
E_LOCAL = 2
TILE = 512
N_TILES = T // TILE


def kernel(x, assign, W1, W2):
    xb = x.astype(jnp.bfloat16)
    w1b = W1.astype(jnp.bfloat16)
    w2b = W2.astype(jnp.bfloat16)
    a2 = assign.reshape(T, 1)

    def body(x_ref, a_ref, w1_ref, w2_ref, out_ref,
             xfull, afull, outfull, recvout, send_sems, recv_sems):
        my_x = lax.axis_index("x")
        my_y = lax.axis_index("y")
        my_z = lax.axis_index("z")
        peer = (1 - my_x, my_y, my_z)

        barrier = pltpu.get_barrier_semaphore()
        pl.semaphore_signal(barrier, inc=1, device_id=peer,
                            device_id_type=pl.DeviceIdType.MESH)
        pl.semaphore_wait(barrier, 1)

        xfull[0] = x_ref[...]
        afull[0] = a_ref[...]

        rx = pltpu.make_async_remote_copy(
            src_ref=xfull.at[0], dst_ref=xfull.at[1],
            send_sem=send_sems.at[0], recv_sem=recv_sems.at[0],
            device_id=peer, device_id_type=pl.DeviceIdType.MESH)
        ra = pltpu.make_async_remote_copy(
            src_ref=afull.at[0], dst_ref=afull.at[1],
            send_sem=send_sems.at[1], recv_sem=recv_sems.at[1],
            device_id=peer, device_id_type=pl.DeviceIdType.MESH)
        rx.start()
        ra.start()
        rx.wait()
        ra.wait()

        e_base = my_x * E_LOCAL
        for s in range(2):
            for t in range(N_TILES):
                rows = pl.ds(t * TILE, TILE)
                xt = xfull[s, rows, :]
                at = afull[s, rows, :]
                acc = jnp.zeros((TILE, D), jnp.float32)
                for le in range(E_LOCAL):
                    h = jnp.dot(xt, w1_ref[le],
                                preferred_element_type=jnp.float32)
                    h = jnp.maximum(h, 0.0).astype(jnp.bfloat16)
                    y = jnp.dot(h, w2_ref[le],
                                preferred_element_type=jnp.float32)
                    mask = (at == e_base + le).astype(jnp.float32)
                    acc = acc + y * mask
                outfull[s, rows, :] = acc.astype(jnp.bfloat16)

        ro = pltpu.make_async_remote_copy(
            src_ref=outfull.at[1], dst_ref=recvout,
            send_sem=send_sems.at[2], recv_sem=recv_sems.at[2],
            device_id=peer, device_id_type=pl.DeviceIdType.MESH)
        ro.start()
        ro.wait()

        out_ref[...] = (outfull[0].astype(jnp.float32)
                        + recvout[...].astype(jnp.float32))

    return pl.pallas_call(
        body,
        out_shape=jax.ShapeDtypeStruct((T, D), jnp.float32),
        in_specs=[pl.BlockSpec(memory_space=pltpu.VMEM)] * 4,
        out_specs=pl.BlockSpec(memory_space=pltpu.VMEM),
        scratch_shapes=[
            pltpu.VMEM((2, T, D), jnp.bfloat16),
            pltpu.VMEM((2, T, 1), jnp.int32),
            pltpu.VMEM((2, T, D), jnp.bfloat16),
            pltpu.VMEM((T, D), jnp.bfloat16),
            pltpu.SemaphoreType.DMA((3,)),
            pltpu.SemaphoreType.DMA((3,)),
        ],
        compiler_params=pltpu.CompilerParams(collective_id=0),
    )(xb, a2, w1b, w2b)


# device time: 90474 ns/iter; 1.2982x vs baseline; 1.2982x over previous
import jax
import jax.numpy as jnp
from jax import lax
from jax.experimental import pallas as pl
from jax.experimental.pallas import tpu as pltpu

T = 1024
D = 1024
F = 2048
E_LOCAL = 2
TILE = 512
N_TILES = T // TILE


def kernel(x, assign, W1, W2):
    xb = x.astype(jnp.bfloat16)
    w1b = W1.astype(jnp.bfloat16)
    w2b = W2.astype(jnp.bfloat16)
    a2 = assign.reshape(T, 1)

    def body(x_ref, a_ref, w1_ref, w2_ref, out_ref,
             xpeer, apeer, outmine, outpeer, recvout, send_sems, recv_sems):
        my_x = lax.axis_index("x")
        my_y = lax.axis_index("y")
        my_z = lax.axis_index("z")
        peer = (1 - my_x, my_y, my_z)

        barrier = pltpu.get_barrier_semaphore()
        pl.semaphore_signal(barrier, inc=1, device_id=peer,
                            device_id_type=pl.DeviceIdType.MESH)
        pl.semaphore_wait(barrier, 1)

        rx = pltpu.make_async_remote_copy(
            src_ref=x_ref, dst_ref=xpeer,
            send_sem=send_sems.at[0], recv_sem=recv_sems.at[0],
            device_id=peer, device_id_type=pl.DeviceIdType.MESH)
        ra = pltpu.make_async_remote_copy(
            src_ref=a_ref, dst_ref=apeer,
            send_sem=send_sems.at[1], recv_sem=recv_sems.at[1],
            device_id=peer, device_id_type=pl.DeviceIdType.MESH)
        rx.start()
        ra.start()

        e_base = my_x * E_LOCAL

        def expert_block(xt, at):
            acc = jnp.zeros((TILE, D), jnp.float32)
            for le in range(E_LOCAL):
                h = jnp.dot(xt, w1_ref[le],
                            preferred_element_type=jnp.float32)
                h = jnp.maximum(h, 0.0).astype(jnp.bfloat16)
                y = jnp.dot(h, w2_ref[le],
                            preferred_element_type=jnp.float32)
                mask = (at == e_base + le).astype(jnp.float32)
                acc = acc + y * mask
            return acc.astype(jnp.bfloat16)

        for t in range(N_TILES):
            rows = pl.ds(t * TILE, TILE)
            outmine[rows, :] = expert_block(x_ref[rows, :], a_ref[rows, :])

        rx.wait()
        ra.wait()

        out_rdmas = []
        for t in range(N_TILES):
            rows = pl.ds(t * TILE, TILE)
            outpeer[rows, :] = expert_block(xpeer[rows, :], apeer[rows, :])
            r = pltpu.make_async_remote_copy(
                src_ref=outpeer.at[rows], dst_ref=recvout.at[rows],
                send_sem=send_sems.at[2 + t], recv_sem=recv_sems.at[2 + t],
                device_id=peer, device_id_type=pl.DeviceIdType.MESH)
            r.start()
            out_rdmas.append(r)
        for r in out_rdmas:
            r.wait()

        out_ref[...] = (outmine[...].astype(jnp.float32)
                        + recvout[...].astype(jnp.float32))

    return pl.pallas_call(
        body,
        out_shape=jax.ShapeDtypeStruct((T, D), jnp.float32),
        in_specs=[pl.BlockSpec(memory_space=pltpu.VMEM)] * 4,
        out_specs=pl.BlockSpec(memory_space=pltpu.VMEM),
        scratch_shapes=[
            pltpu.VMEM((T, D), jnp.bfloat16),
            pltpu.VMEM((T, 1), jnp.int32),
            pltpu.VMEM((T, D), jnp.bfloat16),
            pltpu.VMEM((T, D), jnp.bfloat16),
            pltpu.VMEM((T, D), jnp.bfloat16),
            pltpu.SemaphoreType.DMA((2 + N_TILES,)),
            pltpu.SemaphoreType.DMA((2 + N_TILES,)),
        ],
        compiler_params=pltpu.CompilerParams(collective_id=0),
    )(xb, a2, w1b, w2b)
